# baseline (speedup 1.0000x reference)
import jax
import jax.numpy as jnp
from jax import lax
from jax.experimental import pallas as pl
from jax.experimental.pallas import tpu as pltpu
from jax.experimental.pallas import tpu_sc as plsc

N = 512
NC = 2
NS = 16
L = 16
NW = NC * NS
CHUNK = N // NW


def _sc_body(x_hbm, map_hbm, out_idx_hbm, out_val_hbm,
             x_all, mine_val, mine_map, rank_v, sem):
    w = lax.axis_index("s") * NC + lax.axis_index("c")
    base = w * CHUNK

    pltpu.sync_copy(x_hbm, x_all)
    pltpu.sync_copy(x_hbm.at[pl.ds(base, CHUNK)], mine_val)
    pltpu.sync_copy(map_hbm.at[pl.ds(base, CHUNK)], mine_map)

    iota = lax.iota(jnp.int32, L)
    gidx = iota + base
    m = mine_val[...]

    def chunk_body(t, rank):
        tbase = t * L
        for r in range(L):
            perm = (iota + r) & (L - 1)
            cidx = perm + tbase
            cc = plsc.load_gather(x_all, [cidx])
            before = (cc > m) | ((cc == m) & (cidx < gidx))
            rank = rank + jnp.where(before, 1, 0)
        return rank

    rank = lax.fori_loop(0, NW, chunk_body, jnp.zeros((L,), jnp.int32))
    rank_v[...] = rank

    pltpu.async_copy(mine_val, out_val_hbm.at[rank_v], sem).wait()
    pltpu.async_copy(mine_map, out_idx_hbm.at[rank_v], sem).wait()


@jax.jit
def _sc_sort(x, map_arr):
    call = pl.kernel(
        _sc_body,
        out_type=(
            jax.ShapeDtypeStruct((N,), jnp.int32),
            jax.ShapeDtypeStruct((N,), jnp.float32),
        ),
        mesh=plsc.VectorSubcoreMesh(core_axis_name="c", subcore_axis_name="s"),
        compiler_params=pltpu.CompilerParams(needs_layout_passes=False),
        scratch_types=(
            pltpu.VMEM((N,), jnp.float32),
            pltpu.VMEM((CHUNK,), jnp.float32),
            pltpu.VMEM((CHUNK,), jnp.int32),
            pltpu.VMEM((CHUNK,), jnp.int32),
            pltpu.SemaphoreType.DMA,
        ),
    )
    return call(x, map_arr)


def kernel(input, map_arr):
    return _sc_sort(input, map_arr)
